# Initial kernel scaffold; baseline (speedup 1.0000x reference)
import jax
import jax.numpy as jnp
from jax.experimental import pallas as pl


def kernel(x, edge_index, W1, as1, ad1, b1, g1, be1, W2, as2, ad2, b2, g2, be2, W3, as3, ad3, b3, g3, be3, W4, as4, ad4, b4, pa):
    raise NotImplementedError("write your pallas kernel here")


# speedup vs baseline: 27.1414x; 27.1414x over previous
import functools
import jax
import jax.numpy as jnp
from jax import lax
from jax.experimental import pallas as pl
from jax.experimental.pallas import tpu as pltpu
from jax.experimental.pallas import tpu_sc as plsc

N = 10000
E = 320000
CHUNK = 80
F32 = jnp.float32


@functools.lru_cache(maxsize=None)
def _edge_kernel(H, C, Fp):
    NC, NS = 2, 16
    NW = NC * NS
    steps = E // (NW * CHUNK)
    NJ = Fp // 16
    mesh = plsc.VectorSubcoreMesh(
        core_axis_name="c", subcore_axis_name="s", num_cores=NC)

    @functools.partial(
        pl.kernel,
        mesh=mesh,
        compiler_params=pltpu.CompilerParams(
            needs_layout_passes=False, use_tc_tiling_on_sc=False),
        out_type=[
            jax.ShapeDtypeStruct((NC, N, Fp), F32),
            jax.ShapeDtypeStruct((NC, N, 16), F32),
        ],
        scratch_types=[
            pltpu.VMEM((CHUNK,), jnp.int32),
            pltpu.VMEM((CHUNK,), jnp.int32),
            pltpu.VMEM((CHUNK, 16), F32),
            pltpu.VMEM((CHUNK, 16), F32),
            pltpu.VMEM((CHUNK, Fp), F32),
            pltpu.VMEM((CHUNK, 16), F32),
            pltpu.VMEM((CHUNK, Fp), F32),
            pltpu.VMEM((CHUNK, 16), F32),
            pltpu.VMEM((16,), F32),
            pltpu.VMEM_SHARED((N, Fp), F32),
            pltpu.VMEM_SHARED((N, 16), F32),
            pltpu.SemaphoreType.DMA,
        ],
    )
    def k(xs_hbm, att_hbm, s_hbm, d_hbm, k_hbm, num_out, den_out,
          s_v, d_v, att_s, att_d, xs_rows, ex_buf, zbuf, zbuf16, kv,
          num_sh, den_sh, sem):
        cid = lax.axis_index("c")
        sid = lax.axis_index("s")
        wid = sid * NC + cid
        iota = lax.iota(jnp.int32, 16)

        @pl.when(sid == 0)
        def _init():
            z16 = jnp.zeros((16,), F32)
            def zrow(r, _):
                for j in range(NJ):
                    zbuf[r, pl.ds(16 * j, 16)] = z16
                zbuf16[r, pl.ds(0, 16)] = z16
                return 0
            lax.fori_loop(0, CHUNK, zrow, 0)
            def zc(c, _):
                pltpu.sync_copy(zbuf, num_sh.at[pl.ds(c * CHUNK, CHUNK)])
                pltpu.sync_copy(zbuf16, den_sh.at[pl.ds(c * CHUNK, CHUNK)])
                return 0
            lax.fori_loop(0, N // CHUNK, zc, 0)

        pltpu.sync_copy(k_hbm, kv)
        plsc.subcore_barrier()
        kreg = kv[...]

        def step(t, _):
            base = wid * (E // NW) + t * CHUNK
            pltpu.sync_copy(s_hbm.at[pl.ds(base, CHUNK)], s_v)
            pltpu.sync_copy(d_hbm.at[pl.ds(base, CHUNK)], d_v)
            pltpu.async_copy(att_hbm.at[s_v], att_s, sem).wait()
            pltpu.async_copy(att_hbm.at[d_v], att_d, sem).wait()
            pltpu.async_copy(xs_hbm.at[s_v], xs_rows, sem).wait()

            for g in range(CHUNK // 16):
                ridx = iota + 16 * g
                for h in range(16):
                    if h < H:
                        es16 = plsc.load_gather(
                            att_s, [ridx, jnp.full((16,), h, jnp.int32)])
                        ed16 = plsc.load_gather(
                            att_d, [ridx, jnp.full((16,), 8 + h, jnp.int32)])
                        a = es16 + ed16
                        a = jnp.where(a >= 0, a, 0.2 * a)
                        ex = jnp.exp(a - kreg)
                    else:
                        ex = jnp.zeros((16,), F32)
                    plsc.store_scatter(
                        ex_buf, [ridx, jnp.full((16,), h, jnp.int32)], ex)

            def srow(i, _):
                ifull = jnp.full((16,), 0, jnp.int32) + i
                for j in range(NJ):
                    hm = jnp.minimum((iota + 16 * j) // C, H - 1)
                    sc = plsc.load_gather(ex_buf, [ifull, hm])
                    xs_rows[i, pl.ds(16 * j, 16)] = (
                        xs_rows[i, pl.ds(16 * j, 16)] * sc)
                return 0
            lax.fori_loop(0, CHUNK, srow, 0)

            pltpu.sync_copy(ex_buf, den_sh.at[d_v], add=True)
            pltpu.sync_copy(xs_rows, num_sh.at[d_v], add=True)
            return 0

        lax.fori_loop(0, steps, step, 0)
        plsc.subcore_barrier()

        @pl.when(sid == 0)
        def _out():
            def oc(c, _):
                r = pl.ds(c * CHUNK, CHUNK)
                pltpu.sync_copy(num_sh.at[r], num_out.at[cid, r])
                pltpu.sync_copy(den_sh.at[r], den_out.at[cid, r])
                return 0
            lax.fori_loop(0, N // CHUNK, oc, 0)

    return k


def _logits(xs, a_s, a_d, H, C):
    es = jnp.concatenate(
        [xs[:, h * C:(h + 1) * C] @ jnp.transpose(a_s[h:h + 1])
         for h in range(H)], axis=1)
    ed = jnp.concatenate(
        [xs[:, h * C:(h + 1) * C] @ jnp.transpose(a_d[h:h + 1])
         for h in range(H)], axis=1)
    return es, ed


def _pack_att(es, ed, H):
    z = jnp.zeros((N, 8 - H), F32) if H < 8 else None
    parts = [es, z, ed, z] if H < 8 else [es, ed]
    att = jnp.concatenate(parts, axis=1)
    kv = jnp.full((16,), jnp.maximum(jnp.max(es) + jnp.max(ed), 0.0), F32)
    return att, kv


def _pre1_body(x_ref, w_ref, as_ref, ad_ref, xs_ref, att_ref, k_ref):
    xs = jnp.dot(x_ref[...], w_ref[...], preferred_element_type=F32)
    es, ed = _logits(xs, as_ref[...], ad_ref[...], 8, 16)
    att, kv = _pack_att(es, ed, 8)
    xs_ref[...] = xs
    att_ref[...] = att
    k_ref[...] = kv


def _pre1(x, W1, as1, ad1):
    return pl.pallas_call(
        _pre1_body,
        out_shape=[
            jax.ShapeDtypeStruct((N, 128), F32),
            jax.ShapeDtypeStruct((N, 16), F32),
            jax.ShapeDtypeStruct((16,), F32),
        ],
    )(x, W1, as1, ad1)


def _combine(num, den, H, C, F):
    return jnp.concatenate(
        [num[:, h * C:(h + 1) * C] / (den[:, h:h + 1] + 1e-16)
         for h in range(H)], axis=1)


def _mid_body(H, C, F, Hn, Cn, concat_pad,
              num_ref, den_ref, b_ref, g_ref, be_ref, pa_ref,
              w_ref, as_ref, ad_ref,
              xs_ref, att_ref, k_ref, xsb_ref=None):
    h = _combine(num_ref[...], den_ref[...], H, C, F) + b_ref[...]
    mu = jnp.mean(h, axis=0)
    var = jnp.mean((h - mu) ** 2, axis=0)
    h = (h - mu) / jnp.sqrt(var + 1e-5) * g_ref[...] + be_ref[...]
    h = jnp.where(h >= 0, h, pa_ref[0] * h)
    xs = jnp.dot(h, w_ref[...], preferred_element_type=F32)
    es, ed = _logits(xs, as_ref[...], ad_ref[...], Hn, Cn)
    if concat_pad:
        xs_ref[...] = jnp.concatenate(
            [xs[:, 0:120], jnp.zeros((N, 8), F32)], axis=1)
        xsb_ref[...] = xs[:, 120:200]
        atta, kva = _pack_att(es[:, 0:3], ed[:, 0:3], 3)
        attb, kvb = _pack_att(es[:, 3:5], ed[:, 3:5], 2)
        att_ref[...] = jnp.concatenate([atta[None], attb[None]], axis=0)
        k_ref[...] = jnp.concatenate([kva[None], kvb[None]], axis=0)
    else:
        xs_ref[...] = xs
        att, kv = _pack_att(es, ed, Hn)
        att_ref[...] = att
        k_ref[...] = kv


def _mid(num2, den2, H, C, F, b, g, be, pa, Wn, asn, adn, Hn, Cn, Fn):
    body = functools.partial(_mid_body, H, C, F, Hn, Cn, False)
    return pl.pallas_call(
        body,
        out_shape=[
            jax.ShapeDtypeStruct((N, Fn), F32),
            jax.ShapeDtypeStruct((N, 16), F32),
            jax.ShapeDtypeStruct((16,), F32),
        ],
    )(num2, den2, b, g, be, pa.reshape(1), Wn, asn, adn)


def _mid4(num2, den2, b, g, be, pa, W4, as4, ad4):
    body = functools.partial(_mid_body, 7, 16, 112, 5, 40, True)
    return pl.pallas_call(
        body,
        compiler_params=pltpu.CompilerParams(
            vmem_limit_bytes=100 * 1024 * 1024),
        out_shape=[
            jax.ShapeDtypeStruct((N, 128), F32),
            jax.ShapeDtypeStruct((2, N, 16), F32),
            jax.ShapeDtypeStruct((2, 16), F32),
            jax.ShapeDtypeStruct((N, 80), F32),
        ],
    )(num2, den2, b, g, be, pa.reshape(1), W4, as4, ad4)


def _final_body(numa_ref, dena_ref, numb_ref, denb_ref, b_ref, out_ref):
    outa = _combine(numa_ref[...], dena_ref[...], 3, 40, 120)
    outb = _combine(numb_ref[...], denb_ref[...], 2, 40, 80)
    h = (outa[:, 0:40] + outa[:, 40:80] + outa[:, 80:120]
         + outb[:, 0:40] + outb[:, 40:80]) / 5.0 + b_ref[...]
    m = jnp.max(h, axis=1, keepdims=True)
    ls = m + jnp.log(jnp.sum(jnp.exp(h - m), axis=1, keepdims=True))
    out_ref[...] = h - ls


def _final(numa2, dena2, numb2, denb2, b4):
    return pl.pallas_call(
        _final_body,
        out_shape=jax.ShapeDtypeStruct((N, 40), F32),
    )(numa2, dena2, numb2, denb2, b4)


def kernel(x, edge_index, W1, as1, ad1, b1, g1, be1, W2, as2, ad2, b2, g2,
           be2, W3, as3, ad3, b3, g3, be3, W4, as4, ad4, b4, pa):
    se = edge_index[0]
    de = edge_index[1]

    def run_edge(H, C, Fp, xs, att, kv):
        n2, d2 = _edge_kernel(H, C, Fp)(xs, att, se, de, kv)
        return n2[0] + n2[1], d2[0] + d2[1]

    xs1, att1, k1 = _pre1(x, W1, as1, ad1)
    n1, d1 = run_edge(8, 16, 128, xs1, att1, k1)
    xs2, att2, k2 = _mid(n1, d1, 8, 16, 128, b1, g1, be1, pa,
                         W2, as2, ad2, 7, 16, 112)
    n2, d2 = run_edge(7, 16, 112, xs2, att2, k2)
    xs3, att3, k3 = _mid(n2, d2, 7, 16, 112, b2, g2, be2, pa,
                         W3, as3, ad3, 7, 16, 112)
    n3, d3 = run_edge(7, 16, 112, xs3, att3, k3)
    xs4a, att4, k4, xs4b = _mid4(n3, d3, b3, g3, be3, pa, W4, as4, ad4)
    na, da = run_edge(3, 40, 128, xs4a, att4[0], k4[0])
    nb, db = run_edge(2, 40, 80, xs4b, att4[1], k4[1])
    return _final(na, da, nb, db, b4)
